# baseline (speedup 1.0000x reference)
import jax
import jax.numpy as jnp
import numpy as np
from jax.experimental import pallas as pl


def _perm(Z: int) -> np.ndarray:
    P = np.zeros((4 * Z, 4 * Z), np.float32)
    for p in range(4 * Z):
        j, k, z = p // (2 * Z), (p // Z) % 2, p % Z
        P[p, j * 2 * Z + 2 * z + k] = 1.0
    return P


def _body(x_ref, p_ref, o_ref):
    v = x_ref[0, 0]
    c = jnp.concatenate([v[0], v[1], v[2], v[3]], axis=-1)
    y = jax.lax.dot(c, p_ref[...], precision=jax.lax.Precision.HIGHEST)
    H = o_ref.shape[1]
    W = o_ref.shape[3]
    o_ref[0, :, 0, :, :] = y.reshape(H, W, o_ref.shape[4])


def kernel(x):
    B, Cr3, H, W, Z = x.shape
    C = Cr3 // 8
    G = B * C
    HW = H * W
    xr = x.reshape(G, 2, 4, HW, Z)
    P = jnp.asarray(_perm(Z))

    out = pl.pallas_call(
        _body,
        grid=(G, 2),
        in_specs=[
            pl.BlockSpec((1, 1, 4, HW, Z), lambda g, i: (g, i, 0, 0, 0)),
            pl.BlockSpec((4 * Z, 4 * Z), lambda g, i: (0, 0)),
        ],
        out_specs=pl.BlockSpec((1, H, 1, W, 4 * Z), lambda g, i: (g, 0, i, 0, 0)),
        out_shape=jax.ShapeDtypeStruct((G, H, 2, W, 4 * Z), jnp.float32),
    )(xr, P)
    return out.reshape(B, C, 2 * H, 2 * W, 2 * Z)
